# baseline (speedup 1.0000x reference)
import jax
import jax.numpy as jnp
from jax.experimental import pallas as pl
from jax.experimental.pallas import tpu as pltpu

_NUM_SNPS = 100000
_NUM_IMPACTS = 16
_EMB = 16
_BATCH = 1024

_ROWS = 2000
_GRID = _NUM_SNPS // _ROWS
_NBUF = 2
_HALF = _ROWS // 2


def _head_body(emb_ref, wpt_ref, bp_ref, gamma_ref, beta_ref, wsb_ref,
               bsbb_ref, idx_ref, sb_ref):
    h = jnp.dot(emb_ref[...], wpt_ref[...],
                preferred_element_type=jnp.float32) + bp_ref[...]
    mu = jnp.mean(h, axis=-1, keepdims=True)
    var = jnp.mean((h - mu) ** 2, axis=-1, keepdims=True)
    h = (h - mu) / jnp.sqrt(var + 1e-5) * gamma_ref[...] + beta_ref[...]
    h = jnp.maximum(h, 0.0)
    tab = jnp.dot(h, wsb_ref[...],
                  preferred_element_type=jnp.float32) + bsbb_ref[...]
    idx = idx_ref[...]
    ss = jnp.full(idx.shape, tab[0, 0] * 0.5, jnp.float32)
    bb = jnp.full(idx.shape, tab[0, 1] * 0.5, jnp.float32)
    for k in range(1, _NUM_IMPACTS):
        m = idx == k
        ss = jnp.where(m, tab[k, 0] * 0.5, ss)
        bb = jnp.where(m, tab[k, 1] * 0.5, bb)
    sb_ref[0:1, :] = ss
    sb_ref[1:2, :] = bb


def _dense_body(s_ref, b_ref, x_hbm, o_hbm, xb, ob, insems, outsems):
    j = pl.program_id(0)
    slot = jax.lax.rem(j, _NBUF)

    def fetch(s, t, start):
        for h in range(2):
            op = pltpu.make_async_copy(
                x_hbm.at[pl.ds(t * _ROWS + h * _HALF, _HALF), :],
                xb.at[s, pl.ds(h * _HALF, _HALF), :],
                insems.at[s, h])
            op.start() if start else op.wait()

    def put(s, t, start):
        for h in range(2):
            op = pltpu.make_async_copy(
                ob.at[s, pl.ds(h * _HALF, _HALF), :],
                o_hbm.at[pl.ds(t * _ROWS + h * _HALF, _HALF), :],
                outsems.at[s, h])
            op.start() if start else op.wait()

    @pl.when(j == 0)
    def _():
        fetch(slot, j, True)
        fetch(jax.lax.rem(j + 1, _NBUF), j + 1, True)

    fetch(slot, j, False)

    @pl.when(j >= _NBUF)
    def _():
        put(slot, j - _NBUF, False)

    xx = xb[slot]
    ss = s_ref[...]
    bb = b_ref[...]
    ob[slot] = xx + xx * jnp.tanh(xx * ss + bb)
    put(slot, j, True)

    @pl.when(j + _NBUF < _GRID)
    def _():
        fetch(slot, j + _NBUF, True)

    @pl.when(j == _GRID - 1)
    def _():
        put(jax.lax.rem(j + 1, _NBUF), j - 1, False)
        put(slot, j, False)


def kernel(x, impact_indices, emb, Wp, bp, gamma, beta, ws, bs, wb, bb):
    wpt = Wp.T
    wsb = jnp.concatenate([ws, wb], axis=1)
    bsbb = jnp.concatenate([bs, bb]).reshape(1, 2)
    idx = impact_indices.reshape(1, _NUM_SNPS)

    sb = pl.pallas_call(
        _head_body,
        out_shape=jax.ShapeDtypeStruct((2, _NUM_SNPS), jnp.float32),
    )(emb, wpt, bp.reshape(1, _EMB), gamma.reshape(1, _EMB),
      beta.reshape(1, _EMB), wsb, bsbb, idx)

    s_col = sb[0].reshape(_NUM_SNPS, 1)
    b_col = sb[1].reshape(_NUM_SNPS, 1)
    xt = x.T

    out_t = pl.pallas_call(
        _dense_body,
        grid=(_GRID,),
        in_specs=[
            pl.BlockSpec((_ROWS, 1), lambda j: (j, 0)),
            pl.BlockSpec((_ROWS, 1), lambda j: (j, 0)),
            pl.BlockSpec(memory_space=pl.ANY),
        ],
        out_specs=pl.BlockSpec(memory_space=pl.ANY),
        out_shape=jax.ShapeDtypeStruct((_NUM_SNPS, _BATCH), jnp.float32),
        scratch_shapes=[
            pltpu.VMEM((_NBUF, _ROWS, _BATCH), jnp.float32),
            pltpu.VMEM((_NBUF, _ROWS, _BATCH), jnp.float32),
            pltpu.SemaphoreType.DMA((_NBUF, 2)),
            pltpu.SemaphoreType.DMA((_NBUF, 2)),
        ],
        compiler_params=pltpu.CompilerParams(
            dimension_semantics=("arbitrary",)),
    )(s_col, b_col, xt)
    return out_t.T
